# baseline (speedup 1.0000x reference)
import jax
import jax.numpy as jnp
from jax.experimental import pallas as pl
from jax.experimental.pallas import tpu as pltpu

_K = 16
_TP = 64
_TMV = 112


def _oam_body(xc_ref, out_ref, d_ref, xb_ref, xr_ref, xt_ref):
    nch = xb_ref.shape[0]
    p = xb_ref.shape[2]
    hh = xc_ref.shape[2]

    def convert(ci, _):
        for k in range(_K):
            ac = xc_ref[0, ci * _K + k]
            row = jnp.concatenate(
                [ac[i:i + 1, :] for i in range(hh)], axis=1)
            rb = row.astype(jnp.bfloat16)
            xb_ref[ci, k:k + 1, :] = rb
            xr_ref[ci, k] = jnp.broadcast_to(rb, (_K, p))
        return 0

    jax.lax.fori_loop(0, nch, convert, 0)

    def tconv(ci, _):
        xt_ref[ci] = xb_ref[ci].T
        return 0

    jax.lax.fori_loop(0, nch, tconv, 0)

    tiles = [(i * _TP, _TP) for i in range(p // _TP)]
    if p % _TP:
        tiles.append((p - p % _TP, p % _TP))

    s = jnp.zeros((1, p), jnp.float32)
    for rp0, tp in tiles:
        rep = tp // _K

        def cols_of(ci, rp0=rp0, tp=tp):
            return xt_ref[ci, rp0:rp0 + tp, :]

        def body(ci, carry, rp0=rp0, tp=tp, rep=rep):
            acc, cols = carry
            cols_next = cols_of(jnp.minimum(ci + 1, nch - 1))
            for k in range(_K):
                rowrep = xr_ref[ci, k]
                if rep > 1:
                    rowrep = pltpu.repeat(rowrep, rep, axis=0)
                acc = acc + jnp.abs(cols[:, k:k + 1] - rowrep)
            return acc, cols_next

        acc, _ = jax.lax.fori_loop(
            0, nch, body,
            (jnp.zeros((tp, p), jnp.bfloat16), cols_of(0)))
        accf = acc.astype(jnp.float32)
        d_ref[rp0:rp0 + tp, :] = accf
        s = s + jnp.sum(accf, axis=0, keepdims=True)

    r = 1.0 / jnp.maximum(s, 1e-12)
    u = jnp.full((8, p), 1.0 / p, jnp.float32)
    for _ in range(10):
        acc_u = jnp.zeros((8, p), jnp.float32)
        for t in range(p // _TMV):
            rp0 = t * _TMV
            acc_u = acc_u + jnp.dot(u[:, rp0:rp0 + _TMV],
                                    d_ref[rp0:rp0 + _TMV, :],
                                    preferred_element_type=jnp.float32)
        u = acc_u * r
    for i in range(out_ref.shape[1]):
        w = out_ref.shape[2]
        out_ref[0, i:i + 1, :] = u[0:1, i * w:(i + 1) * w]


def kernel(x):
    b, c, h, w = x.shape
    p = h * w
    out = pl.pallas_call(
        _oam_body,
        grid=(b,),
        in_specs=[pl.BlockSpec((1, c, h, w), lambda i: (i, 0, 0, 0))],
        out_specs=pl.BlockSpec((1, h, w), lambda i: (i, 0, 0)),
        out_shape=jax.ShapeDtypeStruct((b, h, w), jnp.float32),
        scratch_shapes=[pltpu.VMEM((p, p), jnp.float32),
                        pltpu.VMEM((c // _K, _K, p), jnp.bfloat16),
                        pltpu.VMEM((c // _K, _K, _K, p), jnp.bfloat16),
                        pltpu.VMEM((c // _K, p, _K), jnp.bfloat16)],
        compiler_params=pltpu.CompilerParams(
            dimension_semantics=("arbitrary",),
            vmem_limit_bytes=64 * 1024 * 1024,
        ),
    )(x)
    return out


# speedup vs baseline: 1.1542x; 1.1542x over previous
import jax
import jax.numpy as jnp
from jax.experimental import pallas as pl
from jax.experimental.pallas import tpu as pltpu

_K = 16
_TP = 64
_TMV = 112


def _oam_body(xc_ref, out_ref, d_ref, xb_ref, xr_ref):
    nch = xb_ref.shape[0]
    p = xb_ref.shape[2]
    hh = xc_ref.shape[2]

    def convert(ci, _):
        for k in range(_K):
            ac = xc_ref[0, ci * _K + k]
            row = jnp.concatenate(
                [ac[i:i + 1, :] for i in range(hh)], axis=1)
            rb = row.astype(jnp.bfloat16)
            xb_ref[ci, k:k + 1, :] = rb
            xr_ref[ci, k] = jnp.broadcast_to(rb, (_K, p))
        return 0

    jax.lax.fori_loop(0, nch, convert, 0)

    tiles = [(i * _TP, _TP) for i in range(p // _TP)]
    if p % _TP:
        tiles.append((p - p % _TP, p % _TP))

    s = jnp.zeros((1, p), jnp.float32)
    for rp0, tp in tiles:
        rep = tp // _K

        def cols_of(ci, rp0=rp0, tp=tp):
            return xb_ref[ci, :, rp0:rp0 + tp].T

        def body(ci, carry, rp0=rp0, tp=tp, rep=rep):
            acc, cols = carry
            cols_next = cols_of(jnp.minimum(ci + 1, nch - 1))
            for k in range(_K):
                rowrep = xr_ref[ci, k]
                if rep > 1:
                    rowrep = pltpu.repeat(rowrep, rep, axis=0)
                acc = acc + jnp.abs(cols[:, k:k + 1] - rowrep)
            return acc, cols_next

        acc, _ = jax.lax.fori_loop(
            0, nch, body,
            (jnp.zeros((tp, p), jnp.bfloat16), cols_of(0)))
        accf = acc.astype(jnp.float32)
        d_ref[rp0:rp0 + tp, :] = accf
        s = s + jnp.sum(accf, axis=0, keepdims=True)

    r = 1.0 / jnp.maximum(s, 1e-12)
    u = jnp.full((8, p), 1.0 / p, jnp.float32)
    for _ in range(10):
        acc_u = jnp.zeros((8, p), jnp.float32)
        for t in range(p // _TMV):
            rp0 = t * _TMV
            acc_u = acc_u + jnp.dot(u[:, rp0:rp0 + _TMV],
                                    d_ref[rp0:rp0 + _TMV, :],
                                    preferred_element_type=jnp.float32)
        u = acc_u * r
    for i in range(out_ref.shape[1]):
        w = out_ref.shape[2]
        out_ref[0, i:i + 1, :] = u[0:1, i * w:(i + 1) * w]


def kernel(x):
    b, c, h, w = x.shape
    p = h * w
    out = pl.pallas_call(
        _oam_body,
        grid=(b,),
        in_specs=[pl.BlockSpec((1, c, h, w), lambda i: (i, 0, 0, 0))],
        out_specs=pl.BlockSpec((1, h, w), lambda i: (i, 0, 0)),
        out_shape=jax.ShapeDtypeStruct((b, h, w), jnp.float32),
        scratch_shapes=[pltpu.VMEM((p, p), jnp.float32),
                        pltpu.VMEM((c // _K, _K, p), jnp.bfloat16),
                        pltpu.VMEM((c // _K, _K, _K, p), jnp.bfloat16)],
        compiler_params=pltpu.CompilerParams(
            dimension_semantics=("arbitrary",),
            vmem_limit_bytes=64 * 1024 * 1024,
        ),
    )(x)
    return out


# speedup vs baseline: 1.1874x; 1.0288x over previous
import jax
import jax.numpy as jnp
from jax.experimental import pallas as pl
from jax.experimental.pallas import tpu as pltpu

_K = 16
_TP = 64
_TMV = 112


def _oam_body(xc_ref, out_ref, d_ref, xb_ref, xr_ref):
    nch = xb_ref.shape[0]
    p = xb_ref.shape[2]
    hh = xc_ref.shape[2]

    def convert(ci, _):
        for k in range(_K):
            ac = xc_ref[0, ci * _K + k]
            row = jnp.concatenate(
                [ac[i:i + 1, :] for i in range(hh)], axis=1)
            rb = row.astype(jnp.bfloat16)
            xb_ref[ci, k:k + 1, :] = rb
            xr_ref[ci, k] = jnp.broadcast_to(rb, (_K, p))
        return 0

    jax.lax.fori_loop(0, nch, convert, 0)

    tiles = [(i * _TP, _TP) for i in range(p // _TP)]
    if p % _TP:
        tiles.append((p - p % _TP, p % _TP))

    s = jnp.zeros((1, p), jnp.float32)
    for rp0, tp in tiles:
        rep = tp // _K

        def cols_of(ci, rp0=rp0, tp=tp):
            return xb_ref[ci, :, rp0:rp0 + tp].T

        def body(ci, carry, rp0=rp0, tp=tp, rep=rep):
            acc, cols_a, cols_b = carry
            cols_a_next = cols_of(jnp.minimum(2 * ci + 2, nch - 1))
            cols_b_next = cols_of(jnp.minimum(2 * ci + 3, nch - 1))
            for half, cols in ((0, cols_a), (1, cols_b)):
                for k in range(_K):
                    rowrep = xr_ref[2 * ci + half, k]
                    if rep > 1:
                        rowrep = pltpu.repeat(rowrep, rep, axis=0)
                    acc = acc + jnp.abs(cols[:, k:k + 1] - rowrep)
            return acc, cols_a_next, cols_b_next

        acc, _, _ = jax.lax.fori_loop(
            0, nch // 2, body,
            (jnp.zeros((tp, p), jnp.bfloat16), cols_of(0), cols_of(1)))
        accf = acc.astype(jnp.float32)
        d_ref[rp0:rp0 + tp, :] = accf
        s = s + jnp.sum(accf, axis=0, keepdims=True)

    r = 1.0 / jnp.maximum(s, 1e-12)
    u = jnp.full((8, p), 1.0 / p, jnp.float32)
    for _ in range(10):
        acc_u = jnp.zeros((8, p), jnp.float32)
        for t in range(p // _TMV):
            rp0 = t * _TMV
            acc_u = acc_u + jnp.dot(u[:, rp0:rp0 + _TMV],
                                    d_ref[rp0:rp0 + _TMV, :],
                                    preferred_element_type=jnp.float32)
        u = acc_u * r
    for i in range(out_ref.shape[1]):
        w = out_ref.shape[2]
        out_ref[0, i:i + 1, :] = u[0:1, i * w:(i + 1) * w]


def kernel(x):
    b, c, h, w = x.shape
    p = h * w
    out = pl.pallas_call(
        _oam_body,
        grid=(b,),
        in_specs=[pl.BlockSpec((1, c, h, w), lambda i: (i, 0, 0, 0))],
        out_specs=pl.BlockSpec((1, h, w), lambda i: (i, 0, 0)),
        out_shape=jax.ShapeDtypeStruct((b, h, w), jnp.float32),
        scratch_shapes=[pltpu.VMEM((p, p), jnp.float32),
                        pltpu.VMEM((c // _K, _K, p), jnp.bfloat16),
                        pltpu.VMEM((c // _K, _K, _K, p), jnp.bfloat16)],
        compiler_params=pltpu.CompilerParams(
            dimension_semantics=("arbitrary",),
            vmem_limit_bytes=64 * 1024 * 1024,
        ),
    )(x)
    return out


# speedup vs baseline: 1.2629x; 1.0636x over previous
import jax
import jax.numpy as jnp
from jax.experimental import pallas as pl
from jax.experimental.pallas import tpu as pltpu

_K = 16
_TP = 64
_TMV = 112


def _oam_body(xc_ref, out_ref, d_ref, xb_ref, xr_ref):
    nch = xb_ref.shape[0]
    p = xb_ref.shape[2]
    hh = xc_ref.shape[2]

    def convert(ci, _):
        for k in range(_K):
            ac = xc_ref[0, ci * _K + k]
            row = jnp.concatenate(
                [ac[i:i + 1, :] for i in range(hh)], axis=1)
            rb = row.astype(jnp.bfloat16)
            xb_ref[ci, k:k + 1, :] = rb
            xr_ref[ci, k] = jnp.broadcast_to(rb, (_K, p))
        return 0

    jax.lax.fori_loop(0, nch, convert, 0)

    tiles = [(i * _TP, _TP) for i in range(p // _TP)]
    if p % _TP:
        tiles.append((p - p % _TP, p % _TP))

    s = jnp.zeros((1, p), jnp.float32)
    for rp0, tp in tiles:
        rep = tp // _K

        def cols_of(ci, rp0=rp0, tp=tp):
            return xb_ref[ci, :, rp0:rp0 + tp].T

        nu = 4

        def body(ci, carry, rp0=rp0, tp=tp, rep=rep):
            acc = carry[0]
            cols_cur = carry[1:]
            cols_next = tuple(
                cols_of(jnp.minimum(nu * ci + nu + u, nch - 1))
                for u in range(nu))
            for u in range(nu):
                for k in range(_K):
                    rowrep = xr_ref[nu * ci + u, k]
                    if rep > 1:
                        rowrep = pltpu.repeat(rowrep, rep, axis=0)
                    acc = acc + jnp.abs(cols_cur[u][:, k:k + 1] - rowrep)
            return (acc,) + cols_next

        res = jax.lax.fori_loop(
            0, nch // nu, body,
            (jnp.zeros((tp, p), jnp.bfloat16),)
            + tuple(cols_of(u) for u in range(nu)))
        acc = res[0]
        accf = acc.astype(jnp.float32)
        d_ref[rp0:rp0 + tp, :] = accf
        s = s + jnp.sum(accf, axis=0, keepdims=True)

    r = 1.0 / jnp.maximum(s, 1e-12)
    u = jnp.full((8, p), 1.0 / p, jnp.float32)
    for _ in range(10):
        acc_u = jnp.zeros((8, p), jnp.float32)
        for t in range(p // _TMV):
            rp0 = t * _TMV
            acc_u = acc_u + jnp.dot(u[:, rp0:rp0 + _TMV],
                                    d_ref[rp0:rp0 + _TMV, :],
                                    preferred_element_type=jnp.float32)
        u = acc_u * r
    for i in range(out_ref.shape[1]):
        w = out_ref.shape[2]
        out_ref[0, i:i + 1, :] = u[0:1, i * w:(i + 1) * w]


def kernel(x):
    b, c, h, w = x.shape
    p = h * w
    out = pl.pallas_call(
        _oam_body,
        grid=(b,),
        in_specs=[pl.BlockSpec((1, c, h, w), lambda i: (i, 0, 0, 0))],
        out_specs=pl.BlockSpec((1, h, w), lambda i: (i, 0, 0)),
        out_shape=jax.ShapeDtypeStruct((b, h, w), jnp.float32),
        scratch_shapes=[pltpu.VMEM((p, p), jnp.float32),
                        pltpu.VMEM((c // _K, _K, p), jnp.bfloat16),
                        pltpu.VMEM((c // _K, _K, _K, p), jnp.bfloat16)],
        compiler_params=pltpu.CompilerParams(
            dimension_semantics=("arbitrary",),
            vmem_limit_bytes=64 * 1024 * 1024,
        ),
    )(x)
    return out


# speedup vs baseline: 1.2662x; 1.0026x over previous
import jax
import jax.numpy as jnp
from jax.experimental import pallas as pl
from jax.experimental.pallas import tpu as pltpu

_K = 16
_TP = 64
_TMV = 112


def _oam_body(xc_ref, out_ref, d_ref, xb_ref, xr_ref):
    nch = xb_ref.shape[0]
    p = xb_ref.shape[2]
    hh = xc_ref.shape[2]

    def convert(ci, _):
        for k in range(_K):
            ac = xc_ref[0, ci * _K + k]
            row = jnp.concatenate(
                [ac[i:i + 1, :] for i in range(hh)], axis=1)
            rb = row.astype(jnp.bfloat16)
            xb_ref[ci, k:k + 1, :] = rb
            xr_ref[ci, k] = jnp.broadcast_to(rb, (_K, p))
        return 0

    jax.lax.fori_loop(0, nch, convert, 0)

    tiles = [(i * _TP, _TP) for i in range(p // _TP)]
    if p % _TP:
        tiles.append((p - p % _TP, p % _TP))

    s = jnp.zeros((1, p), jnp.float32)
    for rp0, tp in tiles:
        rep = tp // _K

        def cols_of(ci, rp0=rp0, tp=tp):
            return xb_ref[ci, :, rp0:rp0 + tp].T

        nu = 8

        def body(ci, carry, rp0=rp0, tp=tp, rep=rep):
            acc = carry[0]
            cols_cur = carry[1:]
            cols_next = tuple(
                cols_of(jnp.minimum(nu * ci + nu + u, nch - 1))
                for u in range(nu))
            for u in range(nu):
                for k in range(_K):
                    rowrep = xr_ref[nu * ci + u, k]
                    if rep > 1:
                        rowrep = pltpu.repeat(rowrep, rep, axis=0)
                    acc = acc + jnp.abs(cols_cur[u][:, k:k + 1] - rowrep)
            return (acc,) + cols_next

        res = jax.lax.fori_loop(
            0, nch // nu, body,
            (jnp.zeros((tp, p), jnp.bfloat16),)
            + tuple(cols_of(u) for u in range(nu)))
        acc = res[0]
        accf = acc.astype(jnp.float32)
        d_ref[rp0:rp0 + tp, :] = accf
        s = s + jnp.sum(accf, axis=0, keepdims=True)

    r = 1.0 / jnp.maximum(s, 1e-12)
    u = jnp.full((8, p), 1.0 / p, jnp.float32)
    for _ in range(10):
        acc_u = jnp.zeros((8, p), jnp.float32)
        for t in range(p // _TMV):
            rp0 = t * _TMV
            acc_u = acc_u + jnp.dot(u[:, rp0:rp0 + _TMV],
                                    d_ref[rp0:rp0 + _TMV, :],
                                    preferred_element_type=jnp.float32)
        u = acc_u * r
    for i in range(out_ref.shape[1]):
        w = out_ref.shape[2]
        out_ref[0, i:i + 1, :] = u[0:1, i * w:(i + 1) * w]


def kernel(x):
    b, c, h, w = x.shape
    p = h * w
    out = pl.pallas_call(
        _oam_body,
        grid=(b,),
        in_specs=[pl.BlockSpec((1, c, h, w), lambda i: (i, 0, 0, 0))],
        out_specs=pl.BlockSpec((1, h, w), lambda i: (i, 0, 0)),
        out_shape=jax.ShapeDtypeStruct((b, h, w), jnp.float32),
        scratch_shapes=[pltpu.VMEM((p, p), jnp.float32),
                        pltpu.VMEM((c // _K, _K, p), jnp.bfloat16),
                        pltpu.VMEM((c // _K, _K, _K, p), jnp.bfloat16)],
        compiler_params=pltpu.CompilerParams(
            dimension_semantics=("arbitrary",),
            vmem_limit_bytes=64 * 1024 * 1024,
        ),
    )(x)
    return out
